# baseline (speedup 1.0000x reference)
import functools

import jax
import jax.numpy as jnp
from jax import lax
from jax.experimental import pallas as pl
from jax.experimental.pallas import tpu as pltpu
from jax.experimental.pallas import tpu_sc as plsc

B, L, D = 16384, 200, 32
N_TOK = B * L
N_WORKERS = 32
PER_W = N_TOK // N_WORKERS
CHUNK = 1024
SUB = 128
K = CHUNK // SUB
N_ITERS = PER_W // CHUNK
CHAR_ROWS = 1_000_000
LANG_ROWS = 1_000
PROJ_BLK = 16384


TCH = 2048


def _out_t_kernel(x_ref, o_ref):
    x = x_ref[...]
    parts = [x[:, D * u:D * (u + 1)].T for u in range(4)]
    o_ref[...] = jnp.stack(parts, axis=-1).reshape(1, D, TCH)


def _proj_t_kernel(w_ref, xt_ref, o_ref):
    o_ref[...] = lax.dot_general(
        w_ref[...], xt_ref[...], (((1,), (0,)), ((), ())),
        preferred_element_type=jnp.float32)


def _proj_t_bias_kernel(w_ref, xt_ref, b_ref, o_ref):
    o_ref[...] = lax.dot_general(
        w_ref[...], xt_ref[...], (((1,), (0,)), ((), ())),
        preferred_element_type=jnp.float32) + b_ref[...]


def _sc_body(cid_ref, lid_ref, cproj_ref, lproj_ref, out_ref,
             idx_c, idx_l, rows_c, rows_l, sem_c, sem_l):
    nc = 2
    wid = lax.axis_index("s") * nc + lax.axis_index("c")

    def step(g, carry):
        idx = wid * N_ITERS + g
        blk0 = idx * K
        l = idx // (B // CHUNK)
        b0 = (idx % (B // CHUNK)) * CHUNK
        pltpu.sync_copy(cid_ref.at[pl.ds(blk0, K)], idx_c)
        pltpu.sync_copy(lid_ref.at[pl.ds(blk0, K)], idx_l)
        copies = []
        for j in range(K):
            cp = pltpu.make_async_copy(
                cproj_ref.at[idx_c.at[j]], rows_c.at[pl.ds(j * SUB, SUB)],
                sem_c)
            cp.start()
            copies.append(cp)
            cp = pltpu.make_async_copy(
                lproj_ref.at[idx_l.at[j]], rows_l.at[pl.ds(j * SUB, SUB)],
                sem_l)
            cp.start()
            copies.append(cp)
        for cp in copies:
            cp.wait()

        def add_body(i, c2):
            plsc.addupdate(rows_c.at[i, pl.ds(0, 16)], rows_l[i, pl.ds(0, 16)])
            plsc.addupdate(rows_c.at[i, pl.ds(16, 16)], rows_l[i, pl.ds(16, 16)])
            return c2

        lax.fori_loop(0, CHUNK, add_body, 0, unroll=4)
        pltpu.sync_copy(rows_c, out_ref.at[l, pl.ds(b0, CHUNK)])
        return carry

    lax.fori_loop(0, N_ITERS, step, 0)


_sc_gather_add = functools.partial(
    pl.kernel,
    out_type=jax.ShapeDtypeStruct((L, B, D), jnp.float32),
    mesh=plsc.VectorSubcoreMesh(core_axis_name="c", subcore_axis_name="s"),
    scratch_types=[
        pltpu.VMEM((K, SUB), jnp.int32),
        pltpu.VMEM((K, SUB), jnp.int32),
        pltpu.VMEM((CHUNK, D), jnp.float32),
        pltpu.VMEM((CHUNK, D), jnp.float32),
        pltpu.SemaphoreType.DMA,
        pltpu.SemaphoreType.DMA,
    ],
    compiler_params=pltpu.CompilerParams(use_tc_tiling_on_sc=False),
)(_sc_body)


def kernel(char_ids, lang_ids, char_table, lang_table, W, b):
    cid = char_ids.T.astype(jnp.int32).reshape(N_TOK // SUB, SUB)
    lid = lang_ids.T.astype(jnp.int32).reshape(N_TOK // SUB, SUB)

    char_proj_t = pl.pallas_call(
        _proj_t_kernel,
        grid=(pl.cdiv(CHAR_ROWS, PROJ_BLK),),
        in_specs=[
            pl.BlockSpec((D, D), lambda i: (0, 0)),
            pl.BlockSpec((D, PROJ_BLK), lambda i: (0, i)),
        ],
        out_specs=pl.BlockSpec((D, PROJ_BLK), lambda i: (0, i)),
        out_shape=jax.ShapeDtypeStruct((D, CHAR_ROWS), jnp.float32),
    )(W[:, :D], char_table.T)

    lang_proj_t = pl.pallas_call(
        _proj_t_bias_kernel,
        out_shape=jax.ShapeDtypeStruct((D, LANG_ROWS), jnp.float32),
    )(W[:, D:], lang_table.T, b.reshape(D, 1))

    out = _sc_gather_add(cid, lid, char_proj_t.T, lang_proj_t.T)

    packed = out.reshape(N_TOK * D // 128, 128)
    out_t = pl.pallas_call(
        _out_t_kernel,
        grid=(L, B // TCH),
        in_specs=[pl.BlockSpec((TCH // 4, 128),
                               lambda l, j: (l * (B // TCH) + j, 0))],
        out_specs=pl.BlockSpec((1, D, TCH), lambda l, j: (l, 0, j)),
        out_shape=jax.ShapeDtypeStruct((L, D, B), jnp.float32),
    )(packed)
    return out_t.transpose(2, 0, 1)


# speedup vs baseline: 5.5661x; 5.5661x over previous
import functools

import jax
import jax.numpy as jnp
from jax import lax
from jax.experimental import pallas as pl
from jax.experimental.pallas import tpu as pltpu
from jax.experimental.pallas import tpu_sc as plsc

B, L, D = 16384, 200, 32
N_TOK = B * L
N_WORKERS = 32
PER_W = N_TOK // N_WORKERS
CHUNK = 1024
SUB = 128
K = CHUNK // SUB
N_ITERS = PER_W // CHUNK
CHAR_ROWS = 1_000_000
LANG_ROWS = 1_000
PROJ_BLK = 16384


def _proj_t_kernel(w_ref, xt_ref, o_ref):
    o_ref[...] = lax.dot_general(
        w_ref[...], xt_ref[...], (((1,), (0,)), ((), ())),
        preferred_element_type=jnp.float32)


def _proj_t_bias_kernel(w_ref, xt_ref, b_ref, o_ref):
    o_ref[...] = lax.dot_general(
        w_ref[...], xt_ref[...], (((1,), (0,)), ((), ())),
        preferred_element_type=jnp.float32) + b_ref[...]


def _sc_body(cid_ref, lid_ref, cproj_ref, lproj_ref, out_ref,
             idx_c, idx_l, rows_c, rows_l, buf_t, sem_c, sem_l):
    nc = 2
    wid = lax.axis_index("s") * nc + lax.axis_index("c")

    def step(g, carry):
        idx = wid * N_ITERS + g
        blk0 = idx * K
        l = idx // (B // CHUNK)
        b0 = (idx % (B // CHUNK)) * CHUNK
        pltpu.sync_copy(cid_ref.at[pl.ds(blk0, K)], idx_c)
        pltpu.sync_copy(lid_ref.at[pl.ds(blk0, K)], idx_l)
        copies = []
        for j in range(K):
            cp = pltpu.make_async_copy(
                cproj_ref.at[idx_c.at[j]], rows_c.at[pl.ds(j * SUB, SUB)],
                sem_c)
            cp.start()
            copies.append(cp)
            cp = pltpu.make_async_copy(
                lproj_ref.at[idx_l.at[j]], rows_l.at[pl.ds(j * SUB, SUB)],
                sem_l)
            cp.start()
            copies.append(cp)
        for cp in copies:
            cp.wait()

        ramp = lax.iota(jnp.int32, 16)

        def add_body(i, c2):
            col = jnp.full((16,), 0, jnp.int32) + i
            x0 = rows_c[i, pl.ds(0, 16)] + rows_l[i, pl.ds(0, 16)]
            x1 = rows_c[i, pl.ds(16, 16)] + rows_l[i, pl.ds(16, 16)]
            plsc.store_scatter(buf_t, [ramp, col], x0)
            plsc.store_scatter(buf_t, [ramp + 16, col], x1)
            return c2

        lax.fori_loop(0, CHUNK, add_body, 0, unroll=4)
        pltpu.sync_copy(buf_t, out_ref.at[l, :, pl.ds(b0, CHUNK)])
        return carry

    lax.fori_loop(0, N_ITERS, step, 0)


_sc_gather_add = functools.partial(
    pl.kernel,
    out_type=jax.ShapeDtypeStruct((L, D, B), jnp.float32),
    mesh=plsc.VectorSubcoreMesh(core_axis_name="c", subcore_axis_name="s"),
    scratch_types=[
        pltpu.VMEM((K, SUB), jnp.int32),
        pltpu.VMEM((K, SUB), jnp.int32),
        pltpu.VMEM((CHUNK, D), jnp.float32),
        pltpu.VMEM((CHUNK, D), jnp.float32),
        pltpu.VMEM((D, CHUNK), jnp.float32),
        pltpu.SemaphoreType.DMA,
        pltpu.SemaphoreType.DMA,
    ],
    compiler_params=pltpu.CompilerParams(use_tc_tiling_on_sc=False, needs_layout_passes=False),
)(_sc_body)


def kernel(char_ids, lang_ids, char_table, lang_table, W, b):
    cid = char_ids.T.astype(jnp.int32).reshape(N_TOK // SUB, SUB)
    lid = lang_ids.T.astype(jnp.int32).reshape(N_TOK // SUB, SUB)

    char_proj_t = pl.pallas_call(
        _proj_t_kernel,
        grid=(pl.cdiv(CHAR_ROWS, PROJ_BLK),),
        in_specs=[
            pl.BlockSpec((D, D), lambda i: (0, 0)),
            pl.BlockSpec((D, PROJ_BLK), lambda i: (0, i)),
        ],
        out_specs=pl.BlockSpec((D, PROJ_BLK), lambda i: (0, i)),
        out_shape=jax.ShapeDtypeStruct((D, CHAR_ROWS), jnp.float32),
    )(W[:, :D], char_table.T)

    lang_proj_t = pl.pallas_call(
        _proj_t_bias_kernel,
        out_shape=jax.ShapeDtypeStruct((D, LANG_ROWS), jnp.float32),
    )(W[:, D:], lang_table.T, b.reshape(D, 1))

    out_t = _sc_gather_add(cid, lid, char_proj_t.T, lang_proj_t.T)
    return out_t.transpose(2, 0, 1)


# speedup vs baseline: 9.3727x; 1.6839x over previous
import functools

import jax
import jax.numpy as jnp
from jax import lax
from jax.experimental import pallas as pl
from jax.experimental.pallas import tpu as pltpu
from jax.experimental.pallas import tpu_sc as plsc

B, L, D = 16384, 200, 32
N_TOK = B * L
N_WORKERS = 32
PER_W = N_TOK // N_WORKERS
CHUNK = 1024
SUB = 128
K = CHUNK // SUB
N_ITERS = PER_W // CHUNK
CHAR_ROWS = 1_000_000
LANG_ROWS = 1_000
PROJ_BLK = 16384


TCH = 4096


def _unpack_t_kernel(x_ref, o_ref):
    x = x_ref[...]
    eye = (lax.broadcasted_iota(jnp.int32, (128, 128), 0) ==
           lax.broadcasted_iota(jnp.int32, (128, 128), 1)).astype(jnp.float32)
    t = lax.dot_general(eye, x, (((1,), (1,)), ((), ())),
                        preferred_element_type=jnp.float32)
    for u in range(4):
        o_ref[0, :, u * (TCH // 4):(u + 1) * (TCH // 4)] = (
            t[u * D:(u + 1) * D, :])


def _proj_t_kernel(w_ref, xt_ref, o_ref):
    o_ref[...] = lax.dot_general(
        w_ref[...], xt_ref[...], (((1,), (0,)), ((), ())),
        preferred_element_type=jnp.float32)


def _proj_t_bias_kernel(w_ref, xt_ref, b_ref, o_ref):
    o_ref[...] = lax.dot_general(
        w_ref[...], xt_ref[...], (((1,), (0,)), ((), ())),
        preferred_element_type=jnp.float32) + b_ref[...]


def _sc_body(cid_ref, lid_ref, cproj_ref, lproj_ref, out_ref,
             idx_c, idx_l, rows_c, rows_l, sem_c, sem_l):
    nc = 2
    wid = lax.axis_index("s") * nc + lax.axis_index("c")

    def step(g, carry):
        idx = wid * N_ITERS + g
        blk0 = idx * K
        pltpu.sync_copy(cid_ref.at[pl.ds(blk0, K)], idx_c)
        pltpu.sync_copy(lid_ref.at[pl.ds(blk0, K)], idx_l)
        copies = []
        for j in range(K):
            cp = pltpu.make_async_copy(
                cproj_ref.at[idx_c.at[j]], rows_c.at[pl.ds(j * SUB, SUB)],
                sem_c)
            cp.start()
            copies.append(cp)
            cp = pltpu.make_async_copy(
                lproj_ref.at[idx_l.at[j]], rows_l.at[pl.ds(j * SUB, SUB)],
                sem_l)
            cp.start()
            copies.append(cp)
        for cp in copies:
            cp.wait()

        def add_body(i, c2):
            plsc.addupdate(rows_c.at[i, pl.ds(0, 16)], rows_l[i, pl.ds(0, 16)])
            plsc.addupdate(rows_c.at[i, pl.ds(16, 16)], rows_l[i, pl.ds(16, 16)])
            return c2

        lax.fori_loop(0, CHUNK, add_body, 0, unroll=4)
        pltpu.sync_copy(
            rows_c,
            out_ref.at[pl.ds((idx // 4) * CHUNK, CHUNK),
                       pl.ds((idx % 4) * D, D)])
        return carry

    lax.fori_loop(0, N_ITERS, step, 0)


_sc_gather_add = functools.partial(
    pl.kernel,
    out_type=jax.ShapeDtypeStruct((N_TOK * D // 128, 128), jnp.float32),
    mesh=plsc.VectorSubcoreMesh(core_axis_name="c", subcore_axis_name="s"),
    scratch_types=[
        pltpu.VMEM((K, SUB), jnp.int32),
        pltpu.VMEM((K, SUB), jnp.int32),
        pltpu.VMEM((CHUNK, D), jnp.float32),
        pltpu.VMEM((CHUNK, D), jnp.float32),
        pltpu.SemaphoreType.DMA,
        pltpu.SemaphoreType.DMA,
    ],
    compiler_params=pltpu.CompilerParams(use_tc_tiling_on_sc=False, needs_layout_passes=False),
)(_sc_body)


def kernel(char_ids, lang_ids, char_table, lang_table, W, b):
    cid = char_ids.T.astype(jnp.int32).reshape(N_TOK // SUB, SUB)
    lid = lang_ids.T.astype(jnp.int32).reshape(N_TOK // SUB, SUB)

    char_proj_t = pl.pallas_call(
        _proj_t_kernel,
        grid=(pl.cdiv(CHAR_ROWS, PROJ_BLK),),
        in_specs=[
            pl.BlockSpec((D, D), lambda i: (0, 0)),
            pl.BlockSpec((D, PROJ_BLK), lambda i: (0, i)),
        ],
        out_specs=pl.BlockSpec((D, PROJ_BLK), lambda i: (0, i)),
        out_shape=jax.ShapeDtypeStruct((D, CHAR_ROWS), jnp.float32),
    )(W[:, :D], char_table.T)

    lang_proj_t = pl.pallas_call(
        _proj_t_bias_kernel,
        out_shape=jax.ShapeDtypeStruct((D, LANG_ROWS), jnp.float32),
    )(W[:, D:], lang_table.T, b.reshape(D, 1))

    packed = _sc_gather_add(cid, lid, char_proj_t.T, lang_proj_t.T)

    out_t = pl.pallas_call(
        _unpack_t_kernel,
        grid=(L, B // TCH),
        in_specs=[pl.BlockSpec((TCH // 4, 128),
                               lambda l, j: (l * (B // TCH) + j, 0))],
        out_specs=pl.BlockSpec((1, D, TCH), lambda l, j: (l, 0, j)),
        out_shape=jax.ShapeDtypeStruct((L, D, B), jnp.float32),
    )(packed)
    return out_t.transpose(2, 0, 1)
